# baseline (speedup 1.0000x reference)
import functools

import jax
import jax.numpy as jnp
from jax.experimental import pallas as pl
from jax.experimental.pallas import tpu as pltpu

_BH = 256


def _grad_kernel(x_ref, halo_ref, out_ref, *, h_blks):
    i = pl.program_id(0)
    c = pl.program_id(1)
    xb = x_ref[0, 0]

    lw = jnp.concatenate([xb[:, -1:], xb[:, :-1]], axis=1)
    lanes = jax.lax.broadcasted_iota(jnp.int32, xb.shape, 1)
    lw = jnp.where(lanes == 0, 0.0, lw)

    up = jnp.concatenate([xb[-1:, :], xb[:-1, :]], axis=0)
    first = (i % h_blks) == 0
    prev = halo_ref[0, 0, 7:8, :] * jnp.where(first, 0.0, 1.0)
    rows = jax.lax.broadcasted_iota(jnp.int32, xb.shape, 0)
    up = jnp.where(rows == 0, jnp.broadcast_to(prev, xb.shape), up)

    dw = lw - xb
    dh = up - xb
    f = jnp.sqrt(dw * dw + dh * dh)

    @pl.when(c == 0)
    def _():
        out_ref[0] = f

    @pl.when(c > 0)
    def _():
        out_ref[0] = out_ref[0] + f


def _box_kernel(g_ref, a_ref, out_ref):
    ab = a_ref[...]
    g16 = g_ref[0].astype(jnp.bfloat16)
    t = jnp.dot(ab, g16, preferred_element_type=jnp.float32)
    out_ref[0] = jnp.dot(t.astype(jnp.bfloat16), ab,
                         preferred_element_type=jnp.float32)


def kernel(input) -> jnp.ndarray:
    x = input
    b, nc, hdim, wdim = x.shape
    r = wdim // 40
    bh = _BH
    h_blks = hdim // bh

    g = pl.pallas_call(
        functools.partial(_grad_kernel, h_blks=h_blks),
        grid=(b * h_blks, nc),
        in_specs=[
            pl.BlockSpec(
                (1, 1, bh, wdim),
                lambda i, ci: (i // h_blks, ci, i % h_blks, 0),
            ),
            pl.BlockSpec(
                (1, 1, 8, wdim),
                lambda i, ci: (
                    i // h_blks,
                    ci,
                    jnp.maximum((i % h_blks) * (bh // 8) - 1, 0),
                    0,
                ),
            ),
        ],
        out_specs=pl.BlockSpec((1, bh, wdim), lambda i, ci: (i // h_blks, i % h_blks, 0)),
        out_shape=jax.ShapeDtypeStruct((b, hdim, wdim), jnp.float32),
        compiler_params=pltpu.CompilerParams(
            dimension_semantics=("parallel", "arbitrary"),
            vmem_limit_bytes=100 * 1024 * 1024,
        ),
    )(x, x)

    idx = jnp.arange(hdim)
    band = (jnp.abs(idx[:, None] - idx[None, :]) <= r).astype(jnp.bfloat16)

    out = pl.pallas_call(
        _box_kernel,
        grid=(b,),
        in_specs=[
            pl.BlockSpec((1, hdim, wdim), lambda bi: (bi, 0, 0)),
            pl.BlockSpec((hdim, hdim), lambda bi: (0, 0)),
        ],
        out_specs=pl.BlockSpec((1, hdim, wdim), lambda bi: (bi, 0, 0)),
        out_shape=jax.ShapeDtypeStruct((b, hdim, wdim), jnp.float32),
        compiler_params=pltpu.CompilerParams(
            dimension_semantics=("parallel",),
            vmem_limit_bytes=100 * 1024 * 1024,
        ),
    )(g, band)
    return out


# speedup vs baseline: 1.5212x; 1.5212x over previous
import functools

import jax
import jax.numpy as jnp
from jax.experimental import pallas as pl
from jax.experimental.pallas import tpu as pltpu

_BH = 128
_CH = 8


def _grad_kernel(x_ref, halo_ref, out_ref, *, h_blks, ch):
    i = pl.program_id(0)
    cb = pl.program_id(1)
    bh, w = x_ref.shape[2], x_ref.shape[3]

    first = (i % h_blks) == 0
    fscale = jnp.where(first, 0.0, 1.0)
    lanes_row = jax.lax.broadcasted_iota(jnp.int32, (1, w), 1)
    wmask = jnp.where(lanes_row == 0, 0.0, 1.0)
    rows = jax.lax.broadcasted_iota(jnp.int32, (bh, w), 0)

    acc = None
    for ci in range(ch):
        xb = x_ref[0, ci]
        lw = jnp.concatenate([xb[:, -1:], xb[:, :-1]], axis=1) * wmask
        up = jnp.concatenate([xb[-1:, :], xb[:-1, :]], axis=0)
        prev = halo_ref[0, ci, 7:8, :] * fscale
        up = jnp.where(rows == 0, jnp.broadcast_to(prev, (bh, w)), up)
        dw = lw - xb
        dh = up - xb
        f = jnp.sqrt(dw * dw + dh * dh)
        acc = f if acc is None else acc + f

    @pl.when(cb == 0)
    def _():
        out_ref[0] = acc

    @pl.when(cb > 0)
    def _():
        out_ref[0] = out_ref[0] + acc


def _box_kernel(g_ref, a_ref, out_ref):
    ab = a_ref[...]
    g16 = g_ref[0].astype(jnp.bfloat16)
    t = jnp.dot(ab, g16, preferred_element_type=jnp.float32)
    out_ref[0] = jnp.dot(t.astype(jnp.bfloat16), ab,
                         preferred_element_type=jnp.float32)


def kernel(input) -> jnp.ndarray:
    x = input
    b, nc, hdim, wdim = x.shape
    r = wdim // 40
    bh = _BH
    ch = _CH
    h_blks = hdim // bh

    g = pl.pallas_call(
        functools.partial(_grad_kernel, h_blks=h_blks, ch=ch),
        grid=(b * h_blks, nc // ch),
        in_specs=[
            pl.BlockSpec(
                (1, ch, bh, wdim),
                lambda i, cb: (i // h_blks, cb, i % h_blks, 0),
            ),
            pl.BlockSpec(
                (1, ch, 8, wdim),
                lambda i, cb: (
                    i // h_blks,
                    cb,
                    jnp.maximum((i % h_blks) * (bh // 8) - 1, 0),
                    0,
                ),
            ),
        ],
        out_specs=pl.BlockSpec((1, bh, wdim), lambda i, cb: (i // h_blks, i % h_blks, 0)),
        out_shape=jax.ShapeDtypeStruct((b, hdim, wdim), jnp.float32),
        compiler_params=pltpu.CompilerParams(
            dimension_semantics=("parallel", "arbitrary"),
            vmem_limit_bytes=48 * 1024 * 1024,
        ),
    )(x, x)

    idx = jnp.arange(hdim)
    band = (jnp.abs(idx[:, None] - idx[None, :]) <= r).astype(jnp.bfloat16)

    out = pl.pallas_call(
        _box_kernel,
        grid=(b,),
        in_specs=[
            pl.BlockSpec((1, hdim, wdim), lambda bi: (bi, 0, 0)),
            pl.BlockSpec((hdim, hdim), lambda bi: (0, 0)),
        ],
        out_specs=pl.BlockSpec((1, hdim, wdim), lambda bi: (bi, 0, 0)),
        out_shape=jax.ShapeDtypeStruct((b, hdim, wdim), jnp.float32),
        compiler_params=pltpu.CompilerParams(
            dimension_semantics=("parallel",),
            vmem_limit_bytes=48 * 1024 * 1024,
        ),
    )(g, band)
    return out


# speedup vs baseline: 1.7890x; 1.1761x over previous
import functools

import jax
import jax.numpy as jnp
from jax.experimental import pallas as pl
from jax.experimental.pallas import tpu as pltpu

_BH = 128
_CH = 8


def _grad_kernel(x_ref, halo_ref, out_ref, *, h_blks, ch):
    i = pl.program_id(0)
    cb = pl.program_id(1)
    bh, w = x_ref.shape[2], x_ref.shape[3]

    first = (i % h_blks) == 0
    fscale = jnp.where(first, 0.0, 1.0)
    lanes_row = jax.lax.broadcasted_iota(jnp.int32, (1, w), 1)
    wmask = jnp.where(lanes_row == 0, 0.0, 1.0)
    rows = jax.lax.broadcasted_iota(jnp.int32, (bh, w), 0)

    acc = None
    for ci in range(ch):
        xb = x_ref[0, ci]
        lw = jnp.concatenate([xb[:, -1:], xb[:, :-1]], axis=1) * wmask
        up = jnp.concatenate([xb[-1:, :], xb[:-1, :]], axis=0)
        prev = halo_ref[0, ci, 7:8, :] * fscale
        up = jnp.where(rows == 0, jnp.broadcast_to(prev, (bh, w)), up)
        dw = lw - xb
        dh = up - xb
        s = dw * dw + dh * dh
        f = s * jax.lax.rsqrt(s + 1e-30)
        acc = f if acc is None else acc + f

    @pl.when(cb == 0)
    def _():
        out_ref[0] = acc

    @pl.when(cb > 0)
    def _():
        out_ref[0] = out_ref[0] + acc


def _box_kernel(g_ref, a_ref, out_ref):
    ab = a_ref[...]
    g16 = g_ref[0].astype(jnp.bfloat16)
    t = jnp.dot(ab, g16, preferred_element_type=jnp.float32)
    out_ref[0] = jnp.dot(t.astype(jnp.bfloat16), ab,
                         preferred_element_type=jnp.float32)


def kernel(input) -> jnp.ndarray:
    x = input
    b, nc, hdim, wdim = x.shape
    r = wdim // 40
    bh = _BH
    ch = _CH
    h_blks = hdim // bh

    g = pl.pallas_call(
        functools.partial(_grad_kernel, h_blks=h_blks, ch=ch),
        grid=(b * h_blks, nc // ch),
        in_specs=[
            pl.BlockSpec(
                (1, ch, bh, wdim),
                lambda i, cb: (i // h_blks, cb, i % h_blks, 0),
            ),
            pl.BlockSpec(
                (1, ch, 8, wdim),
                lambda i, cb: (
                    i // h_blks,
                    cb,
                    jnp.maximum((i % h_blks) * (bh // 8) - 1, 0),
                    0,
                ),
            ),
        ],
        out_specs=pl.BlockSpec((1, bh, wdim), lambda i, cb: (i // h_blks, i % h_blks, 0)),
        out_shape=jax.ShapeDtypeStruct((b, hdim, wdim), jnp.float32),
        compiler_params=pltpu.CompilerParams(
            dimension_semantics=("parallel", "arbitrary"),
            vmem_limit_bytes=48 * 1024 * 1024,
        ),
    )(x, x)

    idx = jnp.arange(hdim)
    band = (jnp.abs(idx[:, None] - idx[None, :]) <= r).astype(jnp.bfloat16)

    out = pl.pallas_call(
        _box_kernel,
        grid=(b,),
        in_specs=[
            pl.BlockSpec((1, hdim, wdim), lambda bi: (bi, 0, 0)),
            pl.BlockSpec((hdim, hdim), lambda bi: (0, 0)),
        ],
        out_specs=pl.BlockSpec((1, hdim, wdim), lambda bi: (bi, 0, 0)),
        out_shape=jax.ShapeDtypeStruct((b, hdim, wdim), jnp.float32),
        compiler_params=pltpu.CompilerParams(
            dimension_semantics=("parallel",),
            vmem_limit_bytes=48 * 1024 * 1024,
        ),
    )(g, band)
    return out
